# baseline (speedup 1.0000x reference)
import jax
import jax.numpy as jnp
from jax.experimental import pallas as pl
from jax.experimental.pallas import tpu as pltpu

W_IN, C_IN = 214, 7
C1, KW = 14, 5
C2, W2 = 28, 101
HID = 120
G = 8
T = 14
QL = 2 * G + 4
LANES_IN = QL * 8
BT_CONV = 256
BT_FC = 512


def _conv_body(xg_ref, w1_ref, b1_ref, w2_ref, b2_ref, out_ref):
    w1 = w1_ref[...]
    b1 = b1_ref[...]
    w2 = w2_ref[...]
    b2 = b2_ref[...]

    def pool1(t):
        y1 = jnp.dot(xg_ref[:, t * 256:(t + 1) * 256], w1,
                     preferred_element_type=jnp.float32)
        return jnp.maximum(jnp.maximum(y1[:, :128], y1[:, 128:]) + b1,
                           0.0).astype(jnp.bfloat16)

    p_prev = pool1(0)
    for t in range(T):
        p_next = pool1(t + 1) if t + 1 < T else jnp.zeros_like(p_prev)
        cat = jnp.concatenate([p_prev, p_next], axis=1)
        y2 = jnp.dot(cat, w2, preferred_element_type=jnp.float32)
        out_ref[:, t * 256:(t + 1) * 256] = (
            jnp.maximum(y2 + b2, 0.0).astype(jnp.bfloat16))
        p_prev = p_next


def _fc_body(f_ref, wf1_ref, bf1_ref, wf2_ref, bf2_ref, o_ref):
    h = jnp.dot(f_ref[...], wf1_ref[...], preferred_element_type=jnp.float32)
    h = jnp.maximum(h + bf1_ref[...], 0.0)
    o_ref[...] = jnp.sum(h * wf2_ref[...], axis=-1, keepdims=True) + bf2_ref[...]


def _round_up(x, m):
    return -(-x // m) * m


def kernel(x, conv1_w, conv1_b, conv2_w, conv2_b, fc1_w, fc1_b, fc2_w, fc2_b):
    n = x.shape[0]
    n_pad = _round_up(max(n, 1), BT_FC)

    x2d = x[:, :, 0, :]
    if n_pad != n:
        x2d = jnp.pad(x2d, ((0, n_pad - n), (0, 0), (0, 0)))
    x2d = jnp.pad(x2d, ((0, 0), (0, 0), (0, 16 * T + QL - W_IN)))
    idx = 16 * jnp.arange(T)[:, None] + jnp.arange(QL)[None, :]
    xw = x2d[:, :, idx]
    xg = jnp.transpose(xw, (0, 2, 3, 1))
    xg = jnp.pad(xg, ((0, 0), (0, 0), (0, 12), (0, 1)))
    xg = xg.reshape(n_pad, T * 256).astype(jnp.bfloat16)

    w1k = jnp.transpose(conv1_w[:, :, 0, :], (2, 1, 0))
    kq = (jnp.arange(QL)[:, None, None] - jnp.arange(2)[None, :, None]
          - 2 * jnp.arange(G)[None, None, :])
    v1 = jnp.where(((kq >= 0) & (kq < KW))[..., None, None],
                   w1k[jnp.clip(kq, 0, KW - 1)], 0.0)
    w1g = jnp.transpose(v1, (0, 3, 1, 2, 4))
    w1g = jnp.pad(w1g, ((0, 12), (0, 1), (0, 0), (0, 0), (0, 2)))
    w1g = w1g.reshape(256, 256).astype(jnp.bfloat16)
    b1t = jnp.tile(jnp.pad(conv1_b, (0, 2)), G).reshape(1, 128)

    w2k = jnp.transpose(conv2_w[:, :, 0, :], (2, 1, 0))
    kj = jnp.arange(2 * G)[:, None] - jnp.arange(G)[None, :]
    v2 = jnp.where(((kj >= 0) & (kj < KW))[..., None, None],
                   w2k[jnp.clip(kj, 0, KW - 1)], 0.0)
    w2g = jnp.transpose(v2, (0, 2, 1, 3))
    w2g = jnp.pad(w2g, ((0, 0), (0, 2), (0, 0), (0, 4)))
    w2g = w2g.reshape(256, 256).astype(jnp.bfloat16)
    b2t = jnp.tile(jnp.pad(conv2_b, (0, 4)), G).reshape(1, 256)

    rows = BT_CONV * T
    y = pl.pallas_call(
        _conv_body,
        out_shape=jax.ShapeDtypeStruct((n_pad, T * 256), jnp.bfloat16),
        grid=(n_pad // BT_CONV,),
        in_specs=[
            pl.BlockSpec((BT_CONV, T * 256), lambda i: (i, 0)),
            pl.BlockSpec((256, 256), lambda i: (0, 0)),
            pl.BlockSpec((1, 128), lambda i: (0, 0)),
            pl.BlockSpec((256, 256), lambda i: (0, 0)),
            pl.BlockSpec((1, 256), lambda i: (0, 0)),
        ],
        out_specs=pl.BlockSpec((BT_CONV, T * 256), lambda i: (i, 0)),
        compiler_params=pltpu.CompilerParams(dimension_semantics=("parallel",)),
    )(xg, w1g, b1t, w2g, b2t)

    flat = y

    wf = fc1_w.reshape(HID, C2, W2)
    wf = jnp.pad(wf, ((0, 0), (0, 0), (0, G * T - W2)))
    wf = jnp.transpose(wf.reshape(HID, C2, T, G), (2, 3, 1, 0))
    wf = jnp.pad(wf, ((0, 0), (0, 0), (0, 4), (0, 8)))
    wf = wf.reshape(T * 256, 128).astype(jnp.bfloat16)
    bf1p = jnp.pad(fc1_b, (0, 8)).reshape(1, 128)
    wf2p = jnp.pad(fc2_w.reshape(-1), (0, 8)).reshape(1, 128)
    bf2r = fc2_b.reshape(1, 1)

    out = pl.pallas_call(
        _fc_body,
        out_shape=jax.ShapeDtypeStruct((n_pad, 1), jnp.float32),
        grid=(n_pad // BT_FC,),
        in_specs=[
            pl.BlockSpec((BT_FC, T * 256), lambda i: (i, 0)),
            pl.BlockSpec((T * 256, 128), lambda i: (0, 0)),
            pl.BlockSpec((1, 128), lambda i: (0, 0)),
            pl.BlockSpec((1, 128), lambda i: (0, 0)),
            pl.BlockSpec((1, 1), lambda i: (0, 0)),
        ],
        out_specs=pl.BlockSpec((BT_FC, 1), lambda i: (i, 0)),
        compiler_params=pltpu.CompilerParams(dimension_semantics=("parallel",)),
    )(flat, wf, bf1p, wf2p, bf2r)

    return out[:n].reshape(-1)


# speedup vs baseline: 1.7082x; 1.7082x over previous
import jax
import jax.numpy as jnp
from jax.experimental import pallas as pl
from jax.experimental.pallas import tpu as pltpu

W_IN, C_IN = 214, 7
C1, KW = 14, 5
C2, W2 = 28, 101
HID = 120
G = 8
T = 14
T2 = 13
BT = 256


def _body(xg_ref, w1a_ref, w1b_ref, b1_ref, w2_ref, b2_ref, wf_ref,
          bf1_ref, wf2_ref, bf2_ref, o_ref):
    w1a = w1a_ref[...]
    w1b = w1b_ref[...]
    b1 = b1_ref[...]
    w2 = w2_ref[...]
    b2 = b2_ref[...]

    def pool1(t):
        y1 = jnp.dot(xg_ref[:, t * 128:(t + 1) * 128], w1a,
                     preferred_element_type=jnp.float32)
        if t + 1 < T:
            y1 = y1 + jnp.dot(xg_ref[:, (t + 1) * 128:(t + 2) * 128], w1b,
                              preferred_element_type=jnp.float32)
        return jnp.maximum(jnp.maximum(y1[:, :128], y1[:, 128:]) + b1,
                           0.0).astype(jnp.bfloat16)

    p_prev = pool1(0)
    h = jnp.zeros((BT, 128), jnp.float32)
    for t in range(T2):
        p_next = pool1(t + 1)
        cat = jnp.concatenate([p_prev, p_next], axis=1)
        y2 = jnp.dot(cat, w2, preferred_element_type=jnp.float32)
        y2b = jnp.maximum(y2 + b2, 0.0).astype(jnp.bfloat16)
        h = h + jnp.dot(y2b, wf_ref[t * 256:(t + 1) * 256, :],
                        preferred_element_type=jnp.float32)
        p_prev = p_next
    h = jnp.maximum(h + bf1_ref[...], 0.0)
    o_ref[...] = jnp.sum(h * wf2_ref[...], axis=-1, keepdims=True) + bf2_ref[...]


def _round_up(x, m):
    return -(-x // m) * m


def kernel(x, conv1_w, conv1_b, conv2_w, conv2_b, fc1_w, fc1_b, fc2_w, fc2_b):
    n = x.shape[0]
    n_pad = _round_up(max(n, 1), BT)

    x2d = x[:, :, 0, :]
    if n_pad != n:
        x2d = jnp.pad(x2d, ((0, n_pad - n), (0, 0), (0, 0)))
    x2d = jnp.pad(x2d, ((0, 0), (0, 1), (0, 16 * T - W_IN)))
    xa = x2d.reshape(n_pad, 8, T, 16)
    xg = jnp.transpose(xa, (0, 2, 1, 3)).reshape(n_pad, T * 128)
    xg = xg.astype(jnp.bfloat16)

    w1k = jnp.transpose(conv1_w[:, :, 0, :], (2, 1, 0))
    kq = (jnp.arange(2 * G + 4)[:, None, None] - jnp.arange(2)[None, :, None]
          - 2 * jnp.arange(G)[None, None, :])
    v1 = jnp.where(((kq >= 0) & (kq < KW))[..., None, None],
                   w1k[jnp.clip(kq, 0, KW - 1)], 0.0)
    v1 = jnp.transpose(v1, (3, 0, 1, 2, 4))
    v1 = jnp.pad(v1, ((0, 1), (0, 0), (0, 0), (0, 0), (0, 2)))
    w1a = v1[:, :16].reshape(128, 256).astype(jnp.bfloat16)
    w1b = jnp.pad(v1[:, 16:], ((0, 0), (0, 12), (0, 0), (0, 0), (0, 0)))
    w1b = w1b.reshape(128, 256).astype(jnp.bfloat16)
    b1t = jnp.tile(jnp.pad(conv1_b, (0, 2)), G).reshape(1, 128)

    w2k = jnp.transpose(conv2_w[:, :, 0, :], (2, 1, 0))
    kj = jnp.arange(2 * G)[:, None] - jnp.arange(G)[None, :]
    v2 = jnp.where(((kj >= 0) & (kj < KW))[..., None, None],
                   w2k[jnp.clip(kj, 0, KW - 1)], 0.0)
    w2g = jnp.transpose(v2, (0, 2, 1, 3))
    w2g = jnp.pad(w2g, ((0, 0), (0, 2), (0, 0), (0, 4)))
    w2g = w2g.reshape(256, 256).astype(jnp.bfloat16)
    b2t = jnp.tile(jnp.pad(conv2_b, (0, 4)), G).reshape(1, 256)

    wf = fc1_w.reshape(HID, C2, W2)
    wf = jnp.pad(wf, ((0, 0), (0, 0), (0, G * T2 - W2)))
    wf = jnp.transpose(wf.reshape(HID, C2, T2, G), (2, 3, 1, 0))
    wf = jnp.pad(wf, ((0, 0), (0, 0), (0, 4), (0, 8)))
    wf = wf.reshape(T2 * 256, 128).astype(jnp.bfloat16)
    bf1p = jnp.pad(fc1_b, (0, 8)).reshape(1, 128)
    wf2p = jnp.pad(fc2_w.reshape(-1), (0, 8)).reshape(1, 128)
    bf2r = fc2_b.reshape(1, 1)

    out = pl.pallas_call(
        _body,
        out_shape=jax.ShapeDtypeStruct((n_pad, 1), jnp.float32),
        grid=(n_pad // BT,),
        in_specs=[
            pl.BlockSpec((BT, T * 128), lambda i: (i, 0)),
            pl.BlockSpec((128, 256), lambda i: (0, 0)),
            pl.BlockSpec((128, 256), lambda i: (0, 0)),
            pl.BlockSpec((1, 128), lambda i: (0, 0)),
            pl.BlockSpec((256, 256), lambda i: (0, 0)),
            pl.BlockSpec((1, 256), lambda i: (0, 0)),
            pl.BlockSpec((T2 * 256, 128), lambda i: (0, 0)),
            pl.BlockSpec((1, 128), lambda i: (0, 0)),
            pl.BlockSpec((1, 128), lambda i: (0, 0)),
            pl.BlockSpec((1, 1), lambda i: (0, 0)),
        ],
        out_specs=pl.BlockSpec((BT, 1), lambda i: (i, 0)),
        compiler_params=pltpu.CompilerParams(dimension_semantics=("parallel",)),
    )(xg, w1a, w1b, b1t, w2g, b2t, wf, bf1p, wf2p, bf2r)

    return out[:n].reshape(-1)
